# baseline (speedup 1.0000x reference)
import functools

import jax
import jax.numpy as jnp
import numpy as np
from jax import lax
from jax.experimental import pallas as pl
from jax.experimental.pallas import tpu as pltpu
from jax.experimental.pallas import tpu_sc as plsc

N_NODES = 10000
NP = 10240
NC = 2
NS = 16
NW = NC * NS
K = 128
ROWS_PT = NP // NS


def _sc_mesh():
    return plsc.VectorSubcoreMesh(core_axis_name="c", subcore_axis_name="s")


FAST_CORE = 1
STEAL_FRAC = 0.19


@functools.lru_cache(maxsize=None)
def _make_prop(chunks: int, steal: int, d: int, kk: int):
    keep = chunks - steal

    def body(feats_hbm, src_hbm, dst_hbm, out_hbm,
             src_v, dst_v, rows_v, zero_v, acc_sh, gsem):
        c = lax.axis_index("c")
        s = lax.axis_index("s")
        wid = s * NC + c
        cnt = lax.select(c == FAST_CORE,
                         jnp.int32(chunks + steal), jnp.int32(keep))

        z16 = jnp.zeros((16,), jnp.float32)
        for i in range(16):
            for j in range(d // 16):
                zero_v[i, pl.ds(j * 16, 16)] = z16
        base = s * ROWS_PT

        def zbody(k2, carry):
            pltpu.sync_copy(zero_v, acc_sh.at[pl.ds(base + k2 * 16, 16)])
            return carry

        lax.fori_loop(0, ROWS_PT // 16, zbody, 0)

        pltpu.sync_copy(src_hbm.at[wid], src_v.at[pl.ds(0, chunks)])
        pltpu.sync_copy(dst_hbm.at[wid], dst_v.at[pl.ds(0, chunks)])

        @pl.when(c == FAST_CORE)
        def _():
            pw = wid + 1 - 2 * FAST_CORE
            pltpu.sync_copy(src_hbm.at[pw, pl.ds(keep, steal)],
                            src_v.at[pl.ds(chunks, steal)])
            pltpu.sync_copy(dst_hbm.at[pw, pl.ds(keep, steal)],
                            dst_v.at[pl.ds(chunks, steal)])

        plsc.subcore_barrier()

        def lbody(ch, carry):
            pltpu.async_copy(
                feats_hbm.at[src_v.at[ch]], rows_v.at[0], gsem).wait()
            pltpu.sync_copy(rows_v.at[0], acc_sh.at[dst_v.at[ch]], add=True)
            return carry

        lax.fori_loop(0, cnt, lbody, 0)

        plsc.subcore_barrier()
        pltpu.sync_copy(acc_sh.at[pl.ds(base, ROWS_PT)],
                        out_hbm.at[c, pl.ds(base, ROWS_PT)])

    return pl.kernel(
        body,
        out_type=jax.ShapeDtypeStruct((NC, NP, d), jnp.float32),
        mesh=_sc_mesh(),
        scratch_types=[
            pltpu.VMEM((chunks + steal, kk), jnp.int32),
            pltpu.VMEM((chunks + steal, kk), jnp.int32),
            pltpu.VMEM((1, kk, d), jnp.float32),
            pltpu.VMEM((16, d), jnp.float32),
            pltpu.VMEM_SHARED((NP, d), jnp.float32),
            pltpu.SemaphoreType.DMA,
        ],
        compiler_params=pltpu.CompilerParams(use_tc_tiling_on_sc=False),
    )


@functools.lru_cache(maxsize=None)
def _make_deg(chunks: int):
    d = 16

    def body(dst_hbm, out_hbm, dst_v, ones_v, zero_v, acc_sh):
        c = lax.axis_index("c")
        s = lax.axis_index("s")
        wid = s * NC + c
        cnt = jnp.int32(chunks)

        z16 = jnp.zeros((16,), jnp.float32)
        o16 = jnp.ones((16,), jnp.float32)
        for i in range(16):
            zero_v[i, pl.ds(0, 16)] = z16
        for i in range(K):
            ones_v[i, pl.ds(0, 16)] = o16
        base = s * ROWS_PT

        def zbody(k2, carry):
            pltpu.sync_copy(zero_v, acc_sh.at[pl.ds(base + k2 * 16, 16)])
            return carry

        lax.fori_loop(0, ROWS_PT // 16, zbody, 0)
        pltpu.sync_copy(dst_hbm.at[wid], dst_v)
        plsc.subcore_barrier()

        def lbody(ch, carry):
            pltpu.sync_copy(ones_v, acc_sh.at[dst_v.at[ch]], add=True)
            return carry

        lax.fori_loop(0, cnt, lbody, 0)

        plsc.subcore_barrier()
        pltpu.sync_copy(acc_sh.at[pl.ds(base, ROWS_PT)],
                        out_hbm.at[c, pl.ds(base, ROWS_PT)])

    return pl.kernel(
        body,
        out_type=jax.ShapeDtypeStruct((NC, NP, d), jnp.float32),
        mesh=_sc_mesh(),
        scratch_types=[
            pltpu.VMEM((chunks, K), jnp.int32),
            pltpu.VMEM((K, d), jnp.float32),
            pltpu.VMEM((16, d), jnp.float32),
            pltpu.VMEM_SHARED((NP, d), jnp.float32),
        ],
        compiler_params=pltpu.CompilerParams(use_tc_tiling_on_sc=False),
    )


BN = 1000


def _tc_call(body, n, out_specs, out_shapes, in_specs, *args):
    return pl.pallas_call(
        body,
        grid=(n // BN,),
        in_specs=in_specs,
        out_specs=out_specs,
        out_shape=out_shapes,
    )(*args)


def _bs_rows(width):
    return pl.BlockSpec((BN, width), lambda i: (i, 0))


def _bs_pair(width):
    return pl.BlockSpec((NC, BN, width), lambda i: (0, i, 0))


def _bs_full(a, b):
    return pl.BlockSpec((a, b), lambda i: (0, 0))


def _stage_a(degp, x, w1):
    def body(degp_ref, x_ref, w1_ref, dinv_ref, us1_ref):
        deg = degp_ref[0, :, 0:1] + degp_ref[1, :, 0:1] + 1.0
        dinv = lax.rsqrt(deg)
        dinv_ref[...] = dinv
        us1_ref[...] = jnp.dot(x_ref[...], w1_ref[...],
                               preferred_element_type=jnp.float32) * dinv

    n = x.shape[0]
    return _tc_call(
        body, n,
        [_bs_rows(1), _bs_rows(w1.shape[1])],
        [jax.ShapeDtypeStruct((n, 1), jnp.float32),
         jax.ShapeDtypeStruct((n, w1.shape[1]), jnp.float32)],
        [_bs_pair(16), _bs_rows(x.shape[1]), _bs_full(*w1.shape)],
        degp, x, w1)


def _stage_b(us1, p1, dinv, b1, w2):
    def body(us1_ref, p1_ref, dinv_ref, b1_ref, w2_ref, us2_ref):
        dinv = dinv_ref[...]
        h = dinv * (us1_ref[...] + p1_ref[0] + p1_ref[1]) + b1_ref[...]
        h = jnp.maximum(h, 0.0)
        us2_ref[...] = jnp.dot(h, w2_ref[...],
                               preferred_element_type=jnp.float32) * dinv

    n, d = us1.shape
    return _tc_call(
        body, n,
        _bs_rows(w2.shape[1]),
        jax.ShapeDtypeStruct((n, w2.shape[1]), jnp.float32),
        [_bs_rows(d), _bs_pair(d), _bs_rows(1), _bs_full(*b1.shape),
         _bs_full(*w2.shape)],
        us1, p1, dinv, b1, w2)


def _stage_c(us2, p2, dinv, b2):
    def body(us2_ref, p2_ref, dinv_ref, b2_ref, zs_ref):
        dinv = dinv_ref[...]
        z = dinv * (us2_ref[...] + p2_ref[0] + p2_ref[1]) + b2_ref[...]
        zs_ref[...] = z * dinv

    n, d = us2.shape
    return _tc_call(
        body, n,
        _bs_rows(d),
        jax.ShapeDtypeStruct((n, d), jnp.float32),
        [_bs_rows(d), _bs_pair(d), _bs_rows(1), _bs_full(*b2.shape)],
        us2, p2, dinv, b2)


def _stage_d(zs, p3, dinv, w3, b3):
    def body(zs_ref, p3_ref, dinv_ref, w3_ref, b3_ref, ds_ref):
        dinv = dinv_ref[...]
        m3 = dinv * (zs_ref[...] + p3_ref[0] + p3_ref[1])
        dd = jnp.maximum(jnp.dot(m3, w3_ref[...],
                                 preferred_element_type=jnp.float32)
                         + b3_ref[...], 0.0)
        ds_ref[...] = dd * dinv

    n, d = zs.shape
    return _tc_call(
        body, n,
        _bs_rows(w3.shape[1]),
        jax.ShapeDtypeStruct((n, w3.shape[1]), jnp.float32),
        [_bs_rows(d), _bs_pair(d), _bs_rows(1), _bs_full(*w3.shape),
         _bs_full(*b3.shape)],
        zs, p3, dinv, w3, b3)


def _stage_e(ds, p4, dinv, w4, b4):
    def body(ds_ref, p4_ref, dinv_ref, w4_ref, b4_ref, out_ref):
        dinv = dinv_ref[...]
        m4 = dinv * (ds_ref[...] + p4_ref[0] + p4_ref[1])
        out_ref[...] = jnp.dot(m4, w4_ref[...],
                               preferred_element_type=jnp.float32) + b4_ref[...]

    n, d = ds.shape
    return _tc_call(
        body, n,
        _bs_rows(w4.shape[1]),
        jax.ShapeDtypeStruct((n, w4.shape[1]), jnp.float32),
        [_bs_rows(d), _bs_pair(d), _bs_rows(1), _bs_full(*w4.shape),
         _bs_full(*b4.shape)],
        ds, p4, dinv, w4, b4)


def kernel(x, edge_index, W1, b1, W2, b2, W3, b3, W4, b4):
    n = x.shape[0]
    e = edge_index.shape[1]
    e_pad = -(-e // (NW * K)) * (NW * K)
    chunks = e_pad // (NW * K)
    steal = int(round(chunks * STEAL_FRAC))

    src = edge_index[0].astype(jnp.int32)
    dst = edge_index[1].astype(jnp.int32)
    src_r = jnp.concatenate(
        [src, jnp.zeros((e_pad - e,), jnp.int32)]).reshape(NW, chunks, K)
    dst_r = jnp.concatenate(
        [dst, jnp.full((e_pad - e,), n, jnp.int32)]).reshape(NW, chunks, K)
    b1r, b2r, b3r, b4r = (b.reshape(1, -1) for b in (b1, b2, b3, b4))

    degp = _make_deg(chunks)(dst_r)
    dinv, us1 = _stage_a(degp, x, W1)
    p1 = _make_prop(chunks, steal, us1.shape[1], K)(us1, src_r, dst_r)
    us2 = _stage_b(us1, p1, dinv, b1r, W2)
    p2 = _make_prop(chunks, steal, us2.shape[1], K)(us2, src_r, dst_r)
    zs = _stage_c(us2, p2, dinv, b2r)
    p3 = _make_prop(chunks, steal, zs.shape[1], K)(zs, src_r, dst_r)
    ds = _stage_d(zs, p3, dinv, W3, b3r)
    p4 = _make_prop(chunks, steal, ds.shape[1], K)(ds, src_r, dst_r)
    recon = _stage_e(ds, p4, dinv, W4, b4r)
    return recon


# speedup vs baseline: 1.0944x; 1.0944x over previous
import functools

import jax
import jax.numpy as jnp
import numpy as np
from jax import lax
from jax.experimental import pallas as pl
from jax.experimental.pallas import tpu as pltpu
from jax.experimental.pallas import tpu_sc as plsc

N_NODES = 10000
NP = 10240
NC = 2
NS = 16
NW = NC * NS
K = 128
ROWS_PT = NP // NS


def _sc_mesh():
    return plsc.VectorSubcoreMesh(core_axis_name="c", subcore_axis_name="s")


FAST_CORE = 0
STEAL_FRAC = 0.0


@functools.lru_cache(maxsize=None)
def _make_prop(chunks: int, steal: int, d: int, kk: int):
    keep = chunks - steal

    def body(feats_hbm, src_hbm, dst_hbm, out_hbm,
             src_v, dst_v, rows_v, zero_v, acc_sh, gsem):
        c = lax.axis_index("c")
        s = lax.axis_index("s")
        wid = s * NC + c
        cnt = lax.select(c == FAST_CORE,
                         jnp.int32(chunks + steal), jnp.int32(keep))

        z16 = jnp.zeros((16,), jnp.float32)
        for i in range(16):
            for j in range(d // 16):
                zero_v[i, pl.ds(j * 16, 16)] = z16
        base = s * ROWS_PT

        def zbody(k2, carry):
            pltpu.sync_copy(zero_v, acc_sh.at[pl.ds(base + k2 * 16, 16)])
            return carry

        lax.fori_loop(0, ROWS_PT // 16, zbody, 0)

        pltpu.sync_copy(src_hbm.at[wid], src_v.at[pl.ds(0, chunks)])
        pltpu.sync_copy(dst_hbm.at[wid], dst_v.at[pl.ds(0, chunks)])

        if steal:
            @pl.when(c == FAST_CORE)
            def _():
                pw = wid + 1 - 2 * FAST_CORE
                pltpu.sync_copy(src_hbm.at[pw, pl.ds(keep, steal)],
                                src_v.at[pl.ds(chunks, steal)])
                pltpu.sync_copy(dst_hbm.at[pw, pl.ds(keep, steal)],
                                dst_v.at[pl.ds(chunks, steal)])

        plsc.subcore_barrier()

        def lbody(ch, carry):
            pltpu.async_copy(
                feats_hbm.at[src_v.at[ch]], rows_v.at[0], gsem).wait()
            pltpu.sync_copy(rows_v.at[0], acc_sh.at[dst_v.at[ch]], add=True)
            return carry

        lax.fori_loop(0, cnt, lbody, 0)

        plsc.subcore_barrier()
        pltpu.sync_copy(acc_sh.at[pl.ds(base, ROWS_PT)],
                        out_hbm.at[c, pl.ds(base, ROWS_PT)])

    return pl.kernel(
        body,
        out_type=jax.ShapeDtypeStruct((NC, NP, d), jnp.float32),
        mesh=_sc_mesh(),
        scratch_types=[
            pltpu.VMEM((chunks + steal, kk), jnp.int32),
            pltpu.VMEM((chunks + steal, kk), jnp.int32),
            pltpu.VMEM((1, kk, d), jnp.float32),
            pltpu.VMEM((16, d), jnp.float32),
            pltpu.VMEM_SHARED((NP, d), jnp.float32),
            pltpu.SemaphoreType.DMA,
        ],
        compiler_params=pltpu.CompilerParams(use_tc_tiling_on_sc=False),
    )


@functools.lru_cache(maxsize=None)
def _make_deg(chunks: int):
    d = 16

    def body(dst_hbm, out_hbm, dst_v, ones_v, zero_v, acc_sh):
        c = lax.axis_index("c")
        s = lax.axis_index("s")
        wid = s * NC + c
        cnt = jnp.int32(chunks)

        z16 = jnp.zeros((16,), jnp.float32)
        o16 = jnp.ones((16,), jnp.float32)
        for i in range(16):
            zero_v[i, pl.ds(0, 16)] = z16
        for i in range(K):
            ones_v[i, pl.ds(0, 16)] = o16
        base = s * ROWS_PT

        def zbody(k2, carry):
            pltpu.sync_copy(zero_v, acc_sh.at[pl.ds(base + k2 * 16, 16)])
            return carry

        lax.fori_loop(0, ROWS_PT // 16, zbody, 0)
        pltpu.sync_copy(dst_hbm.at[wid], dst_v)
        plsc.subcore_barrier()

        def lbody(ch, carry):
            pltpu.sync_copy(ones_v, acc_sh.at[dst_v.at[ch]], add=True)
            return carry

        lax.fori_loop(0, cnt, lbody, 0)

        plsc.subcore_barrier()
        pltpu.sync_copy(acc_sh.at[pl.ds(base, ROWS_PT)],
                        out_hbm.at[c, pl.ds(base, ROWS_PT)])

    return pl.kernel(
        body,
        out_type=jax.ShapeDtypeStruct((NC, NP, d), jnp.float32),
        mesh=_sc_mesh(),
        scratch_types=[
            pltpu.VMEM((chunks, K), jnp.int32),
            pltpu.VMEM((K, d), jnp.float32),
            pltpu.VMEM((16, d), jnp.float32),
            pltpu.VMEM_SHARED((NP, d), jnp.float32),
        ],
        compiler_params=pltpu.CompilerParams(use_tc_tiling_on_sc=False),
    )


BN = 1000


def _tc_call(body, n, out_specs, out_shapes, in_specs, *args):
    return pl.pallas_call(
        body,
        grid=(n // BN,),
        in_specs=in_specs,
        out_specs=out_specs,
        out_shape=out_shapes,
    )(*args)


def _bs_rows(width):
    return pl.BlockSpec((BN, width), lambda i: (i, 0))


def _bs_pair(width):
    return pl.BlockSpec((NC, BN, width), lambda i: (0, i, 0))


def _bs_full(a, b):
    return pl.BlockSpec((a, b), lambda i: (0, 0))


def _stage_a(degp, x, w1):
    def body(degp_ref, x_ref, w1_ref, dinv_ref, us1_ref):
        deg = degp_ref[0, :, 0:1] + degp_ref[1, :, 0:1] + 1.0
        dinv = lax.rsqrt(deg)
        dinv_ref[...] = dinv
        us1_ref[...] = jnp.dot(x_ref[...], w1_ref[...],
                               preferred_element_type=jnp.float32) * dinv

    n = x.shape[0]
    return _tc_call(
        body, n,
        [_bs_rows(1), _bs_rows(w1.shape[1])],
        [jax.ShapeDtypeStruct((n, 1), jnp.float32),
         jax.ShapeDtypeStruct((n, w1.shape[1]), jnp.float32)],
        [_bs_pair(16), _bs_rows(x.shape[1]), _bs_full(*w1.shape)],
        degp, x, w1)


def _stage_b(us1, p1, dinv, b1, w2):
    def body(us1_ref, p1_ref, dinv_ref, b1_ref, w2_ref, us2_ref):
        dinv = dinv_ref[...]
        h = dinv * (us1_ref[...] + p1_ref[0] + p1_ref[1]) + b1_ref[...]
        h = jnp.maximum(h, 0.0)
        us2_ref[...] = jnp.dot(h, w2_ref[...],
                               preferred_element_type=jnp.float32) * dinv

    n, d = us1.shape
    return _tc_call(
        body, n,
        _bs_rows(w2.shape[1]),
        jax.ShapeDtypeStruct((n, w2.shape[1]), jnp.float32),
        [_bs_rows(d), _bs_pair(d), _bs_rows(1), _bs_full(*b1.shape),
         _bs_full(*w2.shape)],
        us1, p1, dinv, b1, w2)


def _stage_c(us2, p2, dinv, b2):
    def body(us2_ref, p2_ref, dinv_ref, b2_ref, zs_ref):
        dinv = dinv_ref[...]
        z = dinv * (us2_ref[...] + p2_ref[0] + p2_ref[1]) + b2_ref[...]
        zs_ref[...] = z * dinv

    n, d = us2.shape
    return _tc_call(
        body, n,
        _bs_rows(d),
        jax.ShapeDtypeStruct((n, d), jnp.float32),
        [_bs_rows(d), _bs_pair(d), _bs_rows(1), _bs_full(*b2.shape)],
        us2, p2, dinv, b2)


def _stage_d(zs, p3, dinv, w3, b3):
    def body(zs_ref, p3_ref, dinv_ref, w3_ref, b3_ref, ds_ref):
        dinv = dinv_ref[...]
        m3 = dinv * (zs_ref[...] + p3_ref[0] + p3_ref[1])
        dd = jnp.maximum(jnp.dot(m3, w3_ref[...],
                                 preferred_element_type=jnp.float32)
                         + b3_ref[...], 0.0)
        ds_ref[...] = dd * dinv

    n, d = zs.shape
    return _tc_call(
        body, n,
        _bs_rows(w3.shape[1]),
        jax.ShapeDtypeStruct((n, w3.shape[1]), jnp.float32),
        [_bs_rows(d), _bs_pair(d), _bs_rows(1), _bs_full(*w3.shape),
         _bs_full(*b3.shape)],
        zs, p3, dinv, w3, b3)


def _stage_e(ds, p4, dinv, w4, b4):
    def body(ds_ref, p4_ref, dinv_ref, w4_ref, b4_ref, out_ref):
        dinv = dinv_ref[...]
        m4 = dinv * (ds_ref[...] + p4_ref[0] + p4_ref[1])
        out_ref[...] = jnp.dot(m4, w4_ref[...],
                               preferred_element_type=jnp.float32) + b4_ref[...]

    n, d = ds.shape
    return _tc_call(
        body, n,
        _bs_rows(w4.shape[1]),
        jax.ShapeDtypeStruct((n, w4.shape[1]), jnp.float32),
        [_bs_rows(d), _bs_pair(d), _bs_rows(1), _bs_full(*w4.shape),
         _bs_full(*b4.shape)],
        ds, p4, dinv, w4, b4)


def kernel(x, edge_index, W1, b1, W2, b2, W3, b3, W4, b4):
    n = x.shape[0]
    e = edge_index.shape[1]
    e_pad = -(-e // (NW * K)) * (NW * K)
    chunks = e_pad // (NW * K)
    steal = int(round(chunks * STEAL_FRAC))

    src = edge_index[0].astype(jnp.int32)
    dst = edge_index[1].astype(jnp.int32)
    src_r = jnp.concatenate(
        [src, jnp.zeros((e_pad - e,), jnp.int32)]).reshape(NW, chunks, K)
    dst_r = jnp.concatenate(
        [dst, jnp.full((e_pad - e,), n, jnp.int32)]).reshape(NW, chunks, K)
    b1r, b2r, b3r, b4r = (b.reshape(1, -1) for b in (b1, b2, b3, b4))

    degp = _make_deg(chunks)(dst_r)
    dinv, us1 = _stage_a(degp, x, W1)
    p1 = _make_prop(chunks, steal, us1.shape[1], K)(us1, src_r, dst_r)
    us2 = _stage_b(us1, p1, dinv, b1r, W2)
    p2 = _make_prop(chunks, steal, us2.shape[1], K)(us2, src_r, dst_r)
    zs = _stage_c(us2, p2, dinv, b2r)
    p3 = _make_prop(chunks, steal, zs.shape[1], K)(zs, src_r, dst_r)
    ds = _stage_d(zs, p3, dinv, W3, b3r)
    p4 = _make_prop(chunks, steal, ds.shape[1], K)(ds, src_r, dst_r)
    recon = _stage_e(ds, p4, dinv, W4, b4r)
    return recon
